# baseline (speedup 1.0000x reference)
import functools

import jax
import jax.numpy as jnp
from jax import lax
from jax.experimental import pallas as pl
from jax.experimental.pallas import tpu as pltpu
from jax.experimental.pallas import tpu_sc as plsc

_NUM_ROWS = 1000000
_DIM = 32
_BATCH = 16384
_HIST = 50
_NW = 32
_BJ_W = 4
_NBLK = _HIST * _BJ_W

_mesh = plsc.VectorSubcoreMesh(core_axis_name="c", subcore_axis_name="s")


@functools.partial(
    pl.kernel,
    mesh=_mesh,
    out_type=jax.ShapeDtypeStruct((_HIST, 4, 128, 8, 128), jnp.float32),
    scratch_types=[
        pltpu.VMEM((_HIST, 512), jnp.int32),
        pltpu.VMEM((128, _DIM), jnp.float32),
        pltpu.VMEM((128, _DIM), jnp.float32),
        pltpu.VMEM((128, _DIM), jnp.float32),
        pltpu.VMEM((128, _DIM), jnp.float32),
        pltpu.VMEM((_DIM, 128), jnp.float32),
        pltpu.VMEM((_DIM, 128), jnp.float32),
        pltpu.SemaphoreType.DMA,
        pltpu.SemaphoreType.DMA,
        pltpu.SemaphoreType.DMA,
        pltpu.SemaphoreType.DMA,
        pltpu.SemaphoreType.DMA,
        pltpu.SemaphoreType.DMA,
    ],
    compiler_params=pltpu.CompilerParams(
        use_tc_tiling_on_sc=False, needs_layout_passes=False
    ),
)
def _gather_kernel(
    weight_hbm, xt_hbm, out_hbm,
    idx_v, rows0, rows1, rows2, rows3, tblk0, tblk1,
    gsem0, gsem1, gsem2, gsem3, wsem0, wsem1,
):
    wid = lax.axis_index("s") * 2 + lax.axis_index("c")
    col_base = wid * 512
    rows = (rows0, rows1, rows2, rows3)
    tblk = (tblk0, tblk1)
    gsem = (gsem0, gsem1, gsem2, gsem3)
    wsem = (wsem0, wsem1)
    rowvecs = [lax.iota(jnp.int32, 16) + l0 for l0 in range(0, 128, 16)]

    pltpu.sync_copy(xt_hbm.at[:, pl.ds(col_base, 512)], idx_v)

    def fire(k, b):
        h = k % _HIST
        bj = k // _HIST
        pltpu.async_copy(
            weight_hbm.at[idx_v.at[h, pl.ds(bj * 128, 128)]], rows[b], gsem[b]
        )

    def drain(b):
        pltpu.make_async_copy(
            weight_hbm.at[pl.ds(0, 128)], rows[b], gsem[b]
        ).wait()

    def transpose(b, t):
        src = rows[b]
        dst = tblk[t]

        def col_body(c, carry):
            cols = jnp.full((16,), c, jnp.int32)
            for i, rv in enumerate(rowvecs):
                dst[c, pl.ds(i * 16, 16)] = plsc.load_gather(src, [rv, cols])
            return carry

        lax.fori_loop(0, _DIM, col_body, 0)

    def start_wb(k, t):
        h = k % _HIST
        bj = k // _HIST
        for ci in range(4):
            pltpu.async_copy(
                tblk[t].at[pl.ds(ci * 8, 8)],
                out_hbm.at[h, ci, wid * _BJ_W + bj],
                wsem[t],
            )

    def wait_wb(t):
        for ci in range(4):
            pltpu.make_async_copy(
                tblk[t].at[pl.ds(ci * 8, 8)],
                out_hbm.at[0, ci, 0],
                wsem[t],
            ).wait()

    def step(k, b, t, do_wait_wb, do_fire):
        drain(b)
        if do_wait_wb:
            wait_wb(t)
        transpose(b, t)
        if do_fire:
            fire(k + 4, b)
        start_wb(k, t)

    for k in range(4):
        fire(k, k)

    step(0, 0, 0, False, True)
    step(1, 1, 1, False, True)

    def quad_body(p, carry):
        for j in range(4):
            k = 4 * p + 2 + j
            step(k, (2 + j) % 4, j % 2, True, True)
        return carry

    lax.fori_loop(0, (_NBLK - 8) // 4, quad_body, 0)

    for k in range(_NBLK - 6, _NBLK):
        step(k, k % 4, k % 2, True, k + 4 < _NBLK)
    wait_wb(0)
    wait_wb(1)


def kernel(x, weight):
    xt = x.astype(jnp.int32).T
    o5 = _gather_kernel(weight, xt)
    return o5.transpose(2, 4, 0, 1, 3).reshape(_BATCH, _HIST, _DIM)
